# baseline (speedup 1.0000x reference)
import jax
import jax.numpy as jnp
import numpy as np
from jax import lax
from jax.experimental import pallas as pl
from jax.experimental.pallas import tpu as pltpu
from jax.experimental.pallas import tpu_sc as plsc

_NC = 2
_NS = 16
_NW = _NC * _NS
_LANES = 128
_NB = 8
_VEC = 16


def kernel(x, emb_weight):
    batch, hist = x.shape
    vocab1, emb_dim = emb_weight.shape
    pack = _LANES // emb_dim
    batches_per_worker = batch // _NW
    groups_per_worker = batches_per_worker // _NB
    pairs = groups_per_worker // 2

    tbl_lines = emb_weight[:vocab1 - 1].reshape(
        (vocab1 - 1) // pack, _LANES)

    mesh = plsc.VectorSubcoreMesh(core_axis_name="c", subcore_axis_name="s")

    @pl.kernel(
        out_type=jax.ShapeDtypeStruct((batch, hist, _LANES), jnp.float32),
        mesh=mesh,
        scratch_types=[
            pltpu.VMEM((_NB, hist), jnp.int32),
            pltpu.VMEM((_NB, hist), jnp.int32),
            pltpu.VMEM((_NB, hist, _LANES), jnp.float32),
            pltpu.VMEM((_NB, hist, _LANES), jnp.float32),
            pltpu.SemaphoreType.DMA,
            pltpu.SemaphoreType.DMA,
        ],
    )
    def gather_kernel(tbl_hbm, idx_hbm, out_hbm,
                      idx0, idx1, rows0, rows1, gsem, wsem):
        wid = lax.axis_index("s") * _NC + lax.axis_index("c")
        b0 = wid * batches_per_worker

        def run_group(g, idx_v, rows_v):
            b = b0 + g * _NB
            @pl.when(g >= 2)
            def _():
                pltpu.make_async_copy(
                    rows_v, out_hbm.at[pl.ds(b0, _NB)], wsem).wait()

            pltpu.sync_copy(idx_hbm.at[pl.ds(b, _NB)], idx_v)
            copies = [
                pltpu.async_copy(
                    tbl_hbm.at[idx_v.at[j]],
                    rows_v.at[j], gsem)
                for j in range(_NB)
            ]
            for c in copies:
                c.wait()
            pltpu.async_copy(rows_v, out_hbm.at[pl.ds(b, _NB)], wsem)

        @pl.loop(0, pairs)
        def _(p):
            run_group(2 * p, idx0, rows0)
            run_group(2 * p + 1, idx1, rows1)

        pltpu.make_async_copy(rows0, out_hbm.at[pl.ds(b0, _NB)], wsem).wait()
        pltpu.make_async_copy(rows1, out_hbm.at[pl.ds(b0, _NB)], wsem).wait()

    wide3 = gather_kernel(tbl_lines, x >> 2)

    off = (x & (pack - 1))[:, :, None]
    picks = [wide3[:, :, k * emb_dim:(k + 1) * emb_dim] for k in range(pack)]
    out = picks[pack - 1]
    for k in range(pack - 2, -1, -1):
        out = jnp.where(off == k, picks[k], out)
    return out


# speedup vs baseline: 1.8173x; 1.8173x over previous
import jax
import jax.numpy as jnp
from jax import lax
from jax.experimental import pallas as pl
from jax.experimental.pallas import tpu as pltpu
from jax.experimental.pallas import tpu_sc as plsc

_NC = 2
_NS = 16
_NW = _NC * _NS
_LANES = 128
_NB = 8


def kernel(x, emb_weight):
    batch, hist = x.shape
    vocab1, emb_dim = emb_weight.shape
    batches_per_worker = batch // _NW
    groups_per_worker = batches_per_worker // _NB
    pairs = groups_per_worker // 2

    tbl_wide = jnp.pad(emb_weight, ((0, 0), (0, _LANES - emb_dim)))

    mesh = plsc.VectorSubcoreMesh(core_axis_name="c", subcore_axis_name="s")

    @pl.kernel(
        out_type=jax.ShapeDtypeStruct((batch, hist, _LANES), jnp.float32),
        mesh=mesh,
        scratch_types=[
            pltpu.VMEM((_NB, hist), jnp.int32),
            pltpu.VMEM((_NB, hist), jnp.int32),
            pltpu.VMEM((_NB, hist, _LANES), jnp.float32),
            pltpu.VMEM((_NB, hist, _LANES), jnp.float32),
            pltpu.SemaphoreType.DMA,
            pltpu.SemaphoreType.DMA,
        ],
    )
    def gather_kernel(tbl_hbm, idx_hbm, out_hbm,
                      idx0, idx1, rows0, rows1, gsem, wsem):
        wid = lax.axis_index("s") * _NC + lax.axis_index("c")
        b0 = wid * batches_per_worker

        def run_group(g, idx_v, rows_v):
            b = b0 + g * _NB
            @pl.when(g >= 2)
            def _():
                pltpu.make_async_copy(
                    rows_v, out_hbm.at[pl.ds(b0, _NB)], wsem).wait()

            pltpu.sync_copy(idx_hbm.at[pl.ds(b, _NB)], idx_v)
            copies = [
                pltpu.async_copy(
                    tbl_hbm.at[idx_v.at[j]], rows_v.at[j], gsem)
                for j in range(_NB)
            ]
            for c in copies:
                c.wait()
            pltpu.async_copy(rows_v, out_hbm.at[pl.ds(b, _NB)], wsem)

        @pl.loop(0, pairs)
        def _(p):
            run_group(2 * p, idx0, rows0)
            run_group(2 * p + 1, idx1, rows1)

        pltpu.make_async_copy(rows0, out_hbm.at[pl.ds(b0, _NB)], wsem).wait()
        pltpu.make_async_copy(rows1, out_hbm.at[pl.ds(b0, _NB)], wsem).wait()

    wide3 = gather_kernel(tbl_wide, x)
    return wide3[:, :, :emb_dim]


# speedup vs baseline: 1.8174x; 1.0001x over previous
import jax
import jax.numpy as jnp
from jax import lax
from jax.experimental import pallas as pl
from jax.experimental.pallas import tpu as pltpu
from jax.experimental.pallas import tpu_sc as plsc

_NC = 2
_NS = 16
_NW = _NC * _NS
_LANES = 128
_NB = 8


def kernel(x, emb_weight):
    batch, hist = x.shape
    vocab1, emb_dim = emb_weight.shape
    batches_per_worker = batch // _NW
    groups_per_worker = batches_per_worker // _NB
    pairs = groups_per_worker // 2

    tbl_wide = jnp.concatenate(
        [emb_weight,
         jnp.zeros((vocab1, _LANES - emb_dim), jnp.float32)], axis=1)

    mesh = plsc.VectorSubcoreMesh(core_axis_name="c", subcore_axis_name="s")

    @pl.kernel(
        out_type=jax.ShapeDtypeStruct((batch, hist, _LANES), jnp.float32),
        mesh=mesh,
        scratch_types=[
            pltpu.VMEM((_NB, hist), jnp.int32),
            pltpu.VMEM((_NB, hist), jnp.int32),
            pltpu.VMEM((_NB, hist, _LANES), jnp.float32),
            pltpu.VMEM((_NB, hist, _LANES), jnp.float32),
            pltpu.SemaphoreType.DMA,
            pltpu.SemaphoreType.DMA,
        ],
    )
    def gather_kernel(tbl_hbm, idx_hbm, out_hbm,
                      idx0, idx1, rows0, rows1, gsem, wsem):
        wid = lax.axis_index("s") * _NC + lax.axis_index("c")
        b0 = wid * batches_per_worker

        def run_group(g, idx_v, rows_v):
            b = b0 + g * _NB
            @pl.when(g >= 2)
            def _():
                pltpu.make_async_copy(
                    rows_v, out_hbm.at[pl.ds(b0, _NB)], wsem).wait()

            pltpu.sync_copy(idx_hbm.at[pl.ds(b, _NB)], idx_v)
            copies = [
                pltpu.async_copy(
                    tbl_hbm.at[idx_v.at[j]], rows_v.at[j], gsem)
                for j in range(_NB)
            ]
            for c in copies:
                c.wait()
            pltpu.async_copy(rows_v, out_hbm.at[pl.ds(b, _NB)], wsem)

        @pl.loop(0, pairs)
        def _(p):
            run_group(2 * p, idx0, rows0)
            run_group(2 * p + 1, idx1, rows1)

        pltpu.make_async_copy(rows0, out_hbm.at[pl.ds(b0, _NB)], wsem).wait()
        pltpu.make_async_copy(rows1, out_hbm.at[pl.ds(b0, _NB)], wsem).wait()

    wide3 = gather_kernel(tbl_wide, x)
    return wide3[:, :, :emb_dim]
